# Initial kernel scaffold; baseline (speedup 1.0000x reference)
import jax
import jax.numpy as jnp
from jax.experimental import pallas as pl


def kernel(x, edge_index, batch, W1, b1, W2, b2, g1, be1, W3, b3, W4, b4, g2, be2, Wf, bf):
    raise NotImplementedError("write your pallas kernel here")


# speedup vs baseline: 8.3178x; 8.3178x over previous
import jax
import jax.numpy as jnp
from jax import lax
from jax.experimental import pallas as pl
from jax.experimental.pallas import tpu as pltpu
from jax.experimental.pallas import tpu_sc as plsc

N = 10000
E = 320000
G = 64
F = 8
NC = 2
NS = 16
TILES = NC * NS
EPT = E // TILES
CH = 80
NCH = EPT // CH
RPS = N // NS

_HI = lax.Precision.HIGHEST


def _sc_segsum_body(table_hbm, src_hbm, dst_hbm, zeros_hbm, out_hbm,
                    src_v, dst_v, rows_v, acc_sh, sem):
    c = lax.axis_index("c")
    s = lax.axis_index("s")
    wid = c * NS + s

    @pl.when(s == 0)
    def _():
        pltpu.sync_copy(zeros_hbm, acc_sh)

    plsc.subcore_barrier()
    pltpu.sync_copy(src_hbm.at[wid], src_v)
    pltpu.sync_copy(dst_hbm.at[wid], dst_v)

    def chunk(j, carry):
        pltpu.async_copy(table_hbm.at[src_v.at[j]], rows_v, sem).wait()
        pltpu.sync_copy(rows_v, acc_sh.at[dst_v.at[j]], add=True)
        return carry

    lax.fori_loop(0, NCH, chunk, 0)
    plsc.subcore_barrier()

    @pl.when(s == 0)
    def _():
        pltpu.sync_copy(acc_sh, out_hbm.at[c])


_segsum8 = pl.kernel(
    _sc_segsum_body,
    out_type=jax.ShapeDtypeStruct((NC, N, F), jnp.float32),
    mesh=plsc.VectorSubcoreMesh(core_axis_name="c", subcore_axis_name="s"),
    scratch_types=[
        pltpu.VMEM((NCH, CH), jnp.int32),
        pltpu.VMEM((NCH, CH), jnp.int32),
        pltpu.VMEM((CH, F), jnp.float32),
        pltpu.VMEM_SHARED((N, F), jnp.float32),
        pltpu.SemaphoreType.DMA,
    ],
    compiler_params=pltpu.CompilerParams(use_tc_tiling_on_sc=False),
)


def _proj_body(x_ref, w_ref, o_ref):
    o_ref[...] = jnp.dot(x_ref[...], w_ref[...],
                         preferred_element_type=jnp.float32, precision=_HI)


_proj = pl.pallas_call(
    _proj_body, out_shape=jax.ShapeDtypeStruct((N, F), jnp.float32))


def _mlp1_body(u_ref, p_ref, b1_ref, w2_ref, b2_ref, g1_ref, be1_ref, o_ref):
    z = u_ref[...] + p_ref[0] + p_ref[1] + b1_ref[...]
    z = jnp.maximum(z, 0.0)
    z = jnp.dot(z, w2_ref[...], preferred_element_type=jnp.float32,
                precision=_HI) + b2_ref[...]
    z = jnp.maximum(z, 0.0)
    mu = jnp.mean(z, axis=0)
    var = jnp.mean((z - mu) ** 2, axis=0)
    o_ref[...] = (z - mu) / jnp.sqrt(var + 1e-5) * g1_ref[...] + be1_ref[...]


_mlp1 = pl.pallas_call(
    _mlp1_body, out_shape=jax.ShapeDtypeStruct((N, F), jnp.float32))


def _mlp2_body(h_ref, p_ref, w3_ref, b3_ref, w4_ref, b4_ref, g2_ref, be2_ref,
               batch_ref, wf_ref, bf_ref, o_ref, maxp_s):
    z = h_ref[...] + p_ref[0] + p_ref[1]
    z = jnp.maximum(jnp.dot(z, w3_ref[...], preferred_element_type=jnp.float32,
                            precision=_HI) + b3_ref[...], 0.0)
    z = jnp.maximum(jnp.dot(z, w4_ref[...], preferred_element_type=jnp.float32,
                            precision=_HI) + b4_ref[...], 0.0)
    mu = jnp.mean(z, axis=0)
    var = jnp.mean((z - mu) ** 2, axis=0)
    h2 = (z - mu) / jnp.sqrt(var + 1e-5) * g2_ref[...] + be2_ref[...]

    b = batch_ref[...]
    onehot = (b[:, None] == lax.broadcasted_iota(jnp.int32, (N, G), 1)
              ).astype(jnp.float32)
    sump = lax.dot_general(onehot, h2, (((0,), (0,)), ((), ())),
                           preferred_element_type=jnp.float32, precision=_HI)
    counts = jnp.sum(onehot, axis=0)
    meanp = sump / jnp.maximum(counts, 1.0)[:, None]

    def mbody(g, carry):
        row = jnp.max(jnp.where(b[:, None] == g, h2, -jnp.inf), axis=0)
        maxp_s[pl.ds(g, 1), :] = row[None, :]
        return carry

    lax.fori_loop(0, G, mbody, 0)
    pooled = jnp.concatenate([maxp_s[...], meanp], axis=1)
    o_ref[...] = jnp.dot(pooled, wf_ref[...],
                         preferred_element_type=jnp.float32,
                         precision=_HI) + bf_ref[...]


_mlp2 = pl.pallas_call(
    _mlp2_body,
    out_shape=jax.ShapeDtypeStruct((G, 2), jnp.float32),
    scratch_shapes=[pltpu.VMEM((G, 16), jnp.float32)],
)


def kernel(x, edge_index, batch, W1, b1, W2, b2, g1, be1,
           W3, b3, W4, b4, g2, be2, Wf, bf):
    src = edge_index[0].reshape(TILES, NCH, CH)
    dst = edge_index[1].reshape(TILES, NCH, CH)
    zeros8 = jnp.zeros((N, F), jnp.float32)
    u = _proj(x, W1)
    p1 = _segsum8(u, src, dst, zeros8)
    h1 = _mlp1(u, p1, b1, W2, b2, g1, be1)
    p2 = _segsum8(h1, src, dst, zeros8)
    return _mlp2(h1, p2, W3, b3, W4, b4, g2, be2, batch, Wf, bf)


# speedup vs baseline: 10.8500x; 1.3044x over previous
import jax
import jax.numpy as jnp
from jax import lax
from jax.experimental import pallas as pl
from jax.experimental.pallas import tpu as pltpu
from jax.experimental.pallas import tpu_sc as plsc

N = 10000
E = 320000
G = 64
F = 8
NC = 2
NS = 16
TILES = NC * NS
EPT = E // TILES
CH = 80
NCH = EPT // CH
RPS = N // NS

_HI = lax.Precision.HIGHEST


def _sc_segsum_body(table_hbm, src_hbm, dst_hbm, zeros_hbm, out_hbm,
                    src_v, dst_v, rows_a, rows_b, acc_sh, sem_a, sem_b):
    c = lax.axis_index("c")
    s = lax.axis_index("s")
    wid = c * NS + s

    @pl.when(s == 0)
    def _():
        pltpu.sync_copy(zeros_hbm, acc_sh)

    plsc.subcore_barrier()
    pltpu.sync_copy(src_hbm.at[wid], src_v)
    pltpu.sync_copy(dst_hbm.at[wid], dst_v)

    def gat(j, buf, sem):
        return pltpu.make_async_copy(table_hbm.at[src_v.at[j]], buf, sem)

    gat(0, rows_a, sem_a).start()

    def pair(k, carry):
        j0 = 2 * k
        gat(j0 + 1, rows_b, sem_b).start()
        gat(j0, rows_a, sem_a).wait()
        pltpu.sync_copy(rows_a, acc_sh.at[dst_v.at[j0]], add=True)
        gat(j0 + 2, rows_a, sem_a).start()
        gat(j0 + 1, rows_b, sem_b).wait()
        pltpu.sync_copy(rows_b, acc_sh.at[dst_v.at[j0 + 1]], add=True)
        return carry

    lax.fori_loop(0, (NCH - 1) // 2, pair, 0)
    gat(NCH - 1, rows_a, sem_a).wait()
    pltpu.sync_copy(rows_a, acc_sh.at[dst_v.at[NCH - 1]], add=True)
    plsc.subcore_barrier()

    @pl.when(s == 0)
    def _():
        pltpu.sync_copy(acc_sh, out_hbm.at[c])


_segsum8 = pl.kernel(
    _sc_segsum_body,
    out_type=jax.ShapeDtypeStruct((NC, N, F), jnp.float32),
    mesh=plsc.VectorSubcoreMesh(core_axis_name="c", subcore_axis_name="s"),
    scratch_types=[
        pltpu.VMEM((NCH, CH), jnp.int32),
        pltpu.VMEM((NCH, CH), jnp.int32),
        pltpu.VMEM((CH, F), jnp.float32),
        pltpu.VMEM((CH, F), jnp.float32),
        pltpu.VMEM_SHARED((N, F), jnp.float32),
        pltpu.SemaphoreType.DMA,
        pltpu.SemaphoreType.DMA,
    ],
    compiler_params=pltpu.CompilerParams(use_tc_tiling_on_sc=False),
)


def _proj_body(x_ref, w_ref, o_ref):
    o_ref[...] = jnp.dot(x_ref[...], w_ref[...],
                         preferred_element_type=jnp.float32, precision=_HI)


_proj = pl.pallas_call(
    _proj_body, out_shape=jax.ShapeDtypeStruct((N, F), jnp.float32))


def _mlp1_body(u_ref, p_ref, b1_ref, w2_ref, b2_ref, g1_ref, be1_ref, o_ref):
    z = u_ref[...] + p_ref[0] + p_ref[1] + b1_ref[...]
    z = jnp.maximum(z, 0.0)
    z = jnp.dot(z, w2_ref[...], preferred_element_type=jnp.float32,
                precision=_HI) + b2_ref[...]
    z = jnp.maximum(z, 0.0)
    mu = jnp.mean(z, axis=0)
    var = jnp.mean((z - mu) ** 2, axis=0)
    o_ref[...] = (z - mu) / jnp.sqrt(var + 1e-5) * g1_ref[...] + be1_ref[...]


_mlp1 = pl.pallas_call(
    _mlp1_body, out_shape=jax.ShapeDtypeStruct((N, F), jnp.float32))


def _mlp2_body(h_ref, p_ref, w3_ref, b3_ref, w4_ref, b4_ref, g2_ref, be2_ref,
               batch_ref, wf_ref, bf_ref, o_ref, maxp_s):
    z = h_ref[...] + p_ref[0] + p_ref[1]
    z = jnp.maximum(jnp.dot(z, w3_ref[...], preferred_element_type=jnp.float32,
                            precision=_HI) + b3_ref[...], 0.0)
    z = jnp.maximum(jnp.dot(z, w4_ref[...], preferred_element_type=jnp.float32,
                            precision=_HI) + b4_ref[...], 0.0)
    mu = jnp.mean(z, axis=0)
    var = jnp.mean((z - mu) ** 2, axis=0)
    h2 = (z - mu) / jnp.sqrt(var + 1e-5) * g2_ref[...] + be2_ref[...]

    b = batch_ref[...]
    onehot = (b[:, None] == lax.broadcasted_iota(jnp.int32, (N, G), 1)
              ).astype(jnp.float32)
    sump = lax.dot_general(onehot, h2, (((0,), (0,)), ((), ())),
                           preferred_element_type=jnp.float32, precision=_HI)
    counts = jnp.sum(onehot, axis=0)
    meanp = sump / jnp.maximum(counts, 1.0)[:, None]

    def mbody(g, carry):
        row = jnp.max(jnp.where(b[:, None] == g, h2, -jnp.inf), axis=0)
        maxp_s[pl.ds(g, 1), :] = row[None, :]
        return carry

    lax.fori_loop(0, G, mbody, 0)
    pooled = jnp.concatenate([maxp_s[...], meanp], axis=1)
    o_ref[...] = jnp.dot(pooled, wf_ref[...],
                         preferred_element_type=jnp.float32,
                         precision=_HI) + bf_ref[...]


_mlp2 = pl.pallas_call(
    _mlp2_body,
    out_shape=jax.ShapeDtypeStruct((G, 2), jnp.float32),
    scratch_shapes=[pltpu.VMEM((G, 16), jnp.float32)],
)


def kernel(x, edge_index, batch, W1, b1, W2, b2, g1, be1,
           W3, b3, W4, b4, g2, be2, Wf, bf):
    src = edge_index[0].reshape(TILES, NCH, CH)
    dst = edge_index[1].reshape(TILES, NCH, CH)
    zeros8 = jnp.zeros((N, F), jnp.float32)
    u = _proj(x, W1)
    p1 = _segsum8(u, src, dst, zeros8)
    h1 = _mlp1(u, p1, b1, W2, b2, g1, be1)
    p2 = _segsum8(h1, src, dst, zeros8)
    return _mlp2(h1, p2, W3, b3, W4, b4, g2, be2, batch, Wf, bf)


# speedup vs baseline: 13.1320x; 1.2103x over previous
import jax
import jax.numpy as jnp
from jax import lax
from jax.experimental import pallas as pl
from jax.experimental.pallas import tpu as pltpu
from jax.experimental.pallas import tpu_sc as plsc

N = 10000
E = 320000
G = 64
F = 8
NC = 2
NS = 16
TILES = NC * NS
EPT = E // TILES
CH = 80
NCH = EPT // CH
RPS = N // NS

_HI = lax.Precision.HIGHEST


def _sc_segsum_body(table_hbm, src_hbm, dst_hbm, zeros_hbm, out_hbm,
                    src_v, dst_v, rows_a, rows_b, acc_sh, sem_a, sem_b):
    c = lax.axis_index("c")
    s = lax.axis_index("s")
    wid = c * NS + s

    @pl.when(s == 0)
    def _():
        pltpu.sync_copy(zeros_hbm, acc_sh)

    plsc.subcore_barrier()
    pltpu.sync_copy(src_hbm.at[wid], src_v)
    pltpu.sync_copy(dst_hbm.at[wid], dst_v)

    def gat(j, buf, sem):
        return pltpu.make_async_copy(table_hbm.at[src_v.at[j]], buf, sem)

    gat(0, rows_a, sem_a).start()

    def pair(k, carry):
        j0 = 2 * k
        gat(j0 + 1, rows_b, sem_b).start()
        gat(j0, rows_a, sem_a).wait()
        pltpu.sync_copy(rows_a, acc_sh.at[dst_v.at[j0]], add=True)
        gat(j0 + 2, rows_a, sem_a).start()
        gat(j0 + 1, rows_b, sem_b).wait()
        pltpu.sync_copy(rows_b, acc_sh.at[dst_v.at[j0 + 1]], add=True)
        return carry

    lax.fori_loop(0, (NCH - 1) // 2, pair, 0)
    gat(NCH - 1, rows_a, sem_a).wait()
    pltpu.sync_copy(rows_a, acc_sh.at[dst_v.at[NCH - 1]], add=True)
    plsc.subcore_barrier()

    @pl.when(s == 0)
    def _():
        pltpu.sync_copy(acc_sh, out_hbm.at[c])


_segsum8 = pl.kernel(
    _sc_segsum_body,
    out_type=jax.ShapeDtypeStruct((NC, N, F), jnp.float32),
    mesh=plsc.VectorSubcoreMesh(core_axis_name="c", subcore_axis_name="s"),
    scratch_types=[
        pltpu.VMEM((NCH, CH), jnp.int32),
        pltpu.VMEM((NCH, CH), jnp.int32),
        pltpu.VMEM((CH, F), jnp.float32),
        pltpu.VMEM((CH, F), jnp.float32),
        pltpu.VMEM_SHARED((N, F), jnp.float32),
        pltpu.SemaphoreType.DMA,
        pltpu.SemaphoreType.DMA,
    ],
    compiler_params=pltpu.CompilerParams(use_tc_tiling_on_sc=False),
)


def _proj_body(x_ref, w_ref, o_ref):
    o_ref[...] = jnp.dot(x_ref[...], w_ref[...],
                         preferred_element_type=jnp.float32, precision=_HI)


_proj = pl.pallas_call(
    _proj_body, out_shape=jax.ShapeDtypeStruct((N, F), jnp.float32))


def _mlp1_body(u_ref, p_ref, b1_ref, w2_ref, b2_ref, g1_ref, be1_ref, o_ref):
    z = u_ref[...] + p_ref[0] + p_ref[1] + b1_ref[...]
    z = jnp.maximum(z, 0.0)
    z = jnp.dot(z, w2_ref[...], preferred_element_type=jnp.float32,
                precision=_HI) + b2_ref[...]
    z = jnp.maximum(z, 0.0)
    mu = jnp.mean(z, axis=0)
    var = jnp.mean((z - mu) ** 2, axis=0)
    o_ref[...] = (z - mu) / jnp.sqrt(var + 1e-5) * g1_ref[...] + be1_ref[...]


_mlp1 = pl.pallas_call(
    _mlp1_body, out_shape=jax.ShapeDtypeStruct((N, F), jnp.float32))


N_PAD = 10240
TPR = N_PAD // TILES


def _mlp2_body(h_ref, p_ref, w3_ref, b3_ref, w4_ref, b4_ref, g2_ref, be2_ref,
               batch_ref, wf_ref, bf_ref, h2_ref, om_ref):
    z = h_ref[...] + p_ref[0] + p_ref[1]
    z = jnp.maximum(jnp.dot(z, w3_ref[...], preferred_element_type=jnp.float32,
                            precision=_HI) + b3_ref[...], 0.0)
    z = jnp.maximum(jnp.dot(z, w4_ref[...], preferred_element_type=jnp.float32,
                            precision=_HI) + b4_ref[...], 0.0)
    mu = jnp.mean(z, axis=0)
    var = jnp.mean((z - mu) ** 2, axis=0)
    h2 = (z - mu) / jnp.sqrt(var + 1e-5) * g2_ref[...] + be2_ref[...]
    h2_ref[pl.ds(0, N), :] = h2
    h2_ref[pl.ds(N, N_PAD - N), :] = jnp.full((N_PAD - N, 16), -jnp.inf,
                                              jnp.float32)

    b = batch_ref[...]
    onehot = (b[:, None] == lax.broadcasted_iota(jnp.int32, (N, G), 1)
              ).astype(jnp.float32)
    sump = lax.dot_general(onehot, h2, (((0,), (0,)), ((), ())),
                           preferred_element_type=jnp.float32, precision=_HI)
    counts = jnp.sum(onehot, axis=0)
    meanp = sump / jnp.maximum(counts, 1.0)[:, None]
    om_ref[...] = jnp.dot(meanp, wf_ref[pl.ds(16, 16), :],
                          preferred_element_type=jnp.float32,
                          precision=_HI) + bf_ref[...]


_mlp2 = pl.pallas_call(
    _mlp2_body,
    out_shape=(jax.ShapeDtypeStruct((N_PAD, 16), jnp.float32),
               jax.ShapeDtypeStruct((G, 2), jnp.float32)),
)


def _sc_maxpool_body(h_hbm, b_hbm, out_hbm, h_v, b_v, acc_v, tmp_v, stage_sh):
    c = lax.axis_index("c")
    s = lax.axis_index("s")
    wid = c * NS + s
    base = wid * TPR
    pltpu.sync_copy(h_hbm.at[pl.ds(base * 16, TPR * 16)], h_v)
    pltpu.sync_copy(b_hbm.at[pl.ds(base, TPR)], b_v)

    neg = jnp.full((16,), -jnp.inf, jnp.float32)
    lane = lax.iota(jnp.int32, 16)

    def init(i, carry):
        acc_v[pl.ds(i * 16, 16)] = neg
        return carry

    lax.fori_loop(0, G, init, 0)

    def row16(i, carry):
        bb = b_v[pl.ds(i * 16, 16)]
        for k in range(16):
            idx = bb[k] * 16 + lane
            r = h_v[pl.ds((i * 16 + k) * 16, 16)]
            cur = plsc.load_gather(acc_v, [idx])
            plsc.store_scatter(acc_v, [idx], jnp.maximum(cur, r))
        return carry

    lax.fori_loop(0, TPR // 16, row16, 0)
    pltpu.sync_copy(acc_v, stage_sh.at[s])
    plsc.subcore_barrier()

    @pl.when(s == 0)
    def _():
        def comb(k, carry):
            pltpu.sync_copy(stage_sh.at[k], tmp_v)

            def mx(i, c2):
                acc_v[pl.ds(i * 16, 16)] = jnp.maximum(
                    acc_v[pl.ds(i * 16, 16)], tmp_v[pl.ds(i * 16, 16)])
                return c2

            lax.fori_loop(0, G, mx, 0)
            return carry

        lax.fori_loop(1, NS, comb, 0)
        pltpu.sync_copy(acc_v, out_hbm.at[c])


_sc_maxpool = pl.kernel(
    _sc_maxpool_body,
    out_type=jax.ShapeDtypeStruct((NC, G * 16), jnp.float32),
    mesh=plsc.VectorSubcoreMesh(core_axis_name="c", subcore_axis_name="s"),
    scratch_types=[
        pltpu.VMEM((TPR * 16,), jnp.float32),
        pltpu.VMEM((TPR,), jnp.int32),
        pltpu.VMEM((G * 16,), jnp.float32),
        pltpu.VMEM((G * 16,), jnp.float32),
        pltpu.VMEM_SHARED((NS, G * 16), jnp.float32),
    ],
    compiler_params=pltpu.CompilerParams(use_tc_tiling_on_sc=False,
                                         needs_layout_passes=False),
)


def _final_body(mp_ref, wf_ref, om_ref, o_ref):
    mp = jnp.maximum(mp_ref[0], mp_ref[1])
    o_ref[...] = jnp.dot(mp, wf_ref[pl.ds(0, 16), :],
                         preferred_element_type=jnp.float32,
                         precision=_HI) + om_ref[...]


_final = pl.pallas_call(
    _final_body, out_shape=jax.ShapeDtypeStruct((G, 2), jnp.float32))


def kernel(x, edge_index, batch, W1, b1, W2, b2, g1, be1,
           W3, b3, W4, b4, g2, be2, Wf, bf):
    src = edge_index[0].reshape(TILES, NCH, CH)
    dst = edge_index[1].reshape(TILES, NCH, CH)
    zeros8 = jnp.zeros((N, F), jnp.float32)
    batch_pad = jnp.pad(batch, (0, N_PAD - N), constant_values=G - 1)
    u = _proj(x, W1)
    p1 = _segsum8(u, src, dst, zeros8)
    h1 = _mlp1(u, p1, b1, W2, b2, g1, be1)
    p2 = _segsum8(h1, src, dst, zeros8)
    h2, om = _mlp2(h1, p2, W3, b3, W4, b4, g2, be2, batch, Wf, bf)
    mp = _sc_maxpool(h2.reshape(-1), batch_pad)
    return _final(mp.reshape(NC, G, 16), Wf, om)
